# initial kernel scaffold (unmeasured)
import jax
import jax.numpy as jnp
from jax import lax
from jax.experimental import pallas as pl
from jax.experimental.pallas import tpu as pltpu

N_DEV = 32


def kernel(x, W):
    t, d = x.shape
    v_loc = W.shape[1]
    v_glob = N_DEV * v_loc

    def body(x_ref, w_ref, out_ref, chunk_ref, stats_ref, mystats_ref,
             stat_send_sems, stat_recv_sems, ring_send_sems, ring_recv_sems,
             local_sem):
        my = lax.axis_index("i")
        right = lax.rem(my + 1, N_DEV)

        logits = jnp.dot(x_ref[:, :], w_ref[:, :],
                         preferred_element_type=jnp.float32)
        m_loc = jnp.max(logits, axis=1)
        e = jnp.exp(logits - m_loc[:, None])
        s_loc = jnp.sum(e, axis=1)
        chunk_ref[:, :] = e
        mystats_ref[0, :] = m_loc
        mystats_ref[1, :] = s_loc

        for dlt in range(1, N_DEV):
            tgt = lax.rem(my + dlt, N_DEV)
            pltpu.make_async_remote_copy(
                src_ref=mystats_ref,
                dst_ref=stats_ref.at[my],
                send_sem=stat_send_sems.at[dlt - 1],
                recv_sem=stat_recv_sems.at[dlt - 1],
                device_id=(tgt,),
                device_id_type=pl.DeviceIdType.MESH,
            ).start()
        cp = pltpu.make_async_copy(mystats_ref, stats_ref.at[my], local_sem)
        cp.start()
        cp.wait()
        for dlt in range(1, N_DEV):
            src = lax.rem(my - dlt + N_DEV, N_DEV)
            pltpu.make_async_remote_copy(
                src_ref=mystats_ref,
                dst_ref=stats_ref.at[src],
                send_sem=stat_send_sems.at[dlt - 1],
                recv_sem=stat_recv_sems.at[dlt - 1],
                device_id=(my,),
                device_id_type=pl.DeviceIdType.MESH,
            ).wait_recv()
        for dlt in range(1, N_DEV):
            tgt = lax.rem(my + dlt, N_DEV)
            pltpu.make_async_remote_copy(
                src_ref=mystats_ref,
                dst_ref=stats_ref.at[my],
                send_sem=stat_send_sems.at[dlt - 1],
                recv_sem=stat_recv_sems.at[dlt - 1],
                device_id=(tgt,),
                device_id_type=pl.DeviceIdType.MESH,
            ).wait_send()

        allm = stats_ref[:, 0, :]
        alls = stats_ref[:, 1, :]
        gm = jnp.max(allm, axis=0)
        gs = jnp.sum(alls * jnp.exp(allm - gm[None, :]), axis=0)
        scale = jnp.exp(m_loc - gm) / gs
        chunk_ref[:, :] = chunk_ref[:, :] * scale[:, None]

        cp2 = pltpu.make_async_copy(
            chunk_ref, out_ref.at[:, pl.ds(my * v_loc, v_loc)], local_sem)
        cp2.start()
        cp2.wait()

        for h in range(N_DEV - 1):
            o_s = lax.rem(my - h + 2 * N_DEV, N_DEV)
            o_r = lax.rem(my - h - 1 + 2 * N_DEV, N_DEV)
            send = pltpu.make_async_remote_copy(
                src_ref=out_ref.at[:, pl.ds(o_s * v_loc, v_loc)],
                dst_ref=out_ref.at[:, pl.ds(o_s * v_loc, v_loc)],
                send_sem=ring_send_sems.at[h],
                recv_sem=ring_recv_sems.at[h],
                device_id=(right,),
                device_id_type=pl.DeviceIdType.MESH,
            )
            send.start()
            send.wait_send()
            recv = pltpu.make_async_remote_copy(
                src_ref=out_ref.at[:, pl.ds(o_r * v_loc, v_loc)],
                dst_ref=out_ref.at[:, pl.ds(o_r * v_loc, v_loc)],
                send_sem=ring_send_sems.at[h],
                recv_sem=ring_recv_sems.at[h],
                device_id=(right,),
                device_id_type=pl.DeviceIdType.MESH,
            )
            recv.wait_recv()

    return pl.pallas_call(
        body,
        out_shape=jax.ShapeDtypeStruct((t, v_glob), jnp.float32),
        in_specs=[
            pl.BlockSpec(memory_space=pltpu.VMEM),
            pl.BlockSpec(memory_space=pltpu.VMEM),
        ],
        out_specs=pl.BlockSpec(memory_space=pltpu.ANY),
        scratch_shapes=[
            pltpu.VMEM((t, v_loc), jnp.float32),
            pltpu.VMEM((N_DEV, 2, t), jnp.float32),
            pltpu.VMEM((2, t), jnp.float32),
            pltpu.SemaphoreType.DMA((N_DEV - 1,)),
            pltpu.SemaphoreType.DMA((N_DEV - 1,)),
            pltpu.SemaphoreType.DMA((N_DEV - 1,)),
            pltpu.SemaphoreType.DMA((N_DEV - 1,)),
            pltpu.SemaphoreType.DMA,
        ],
    )(x, W)


# baseline (device time: 1577911 ns/iter reference)
import jax
import jax.numpy as jnp
from jax import lax
from jax.experimental import pallas as pl
from jax.experimental.pallas import tpu as pltpu

N_DEV = 32


def kernel(x, W):
    t, d = x.shape
    v_loc = W.shape[1]
    v_glob = N_DEV * v_loc

    def body(x_ref, w_ref, out_ref, chunk_ref, stats_ref, mystats_ref,
             stat_send_sems, stat_recv_sems, ring_send_sems, ring_recv_sems,
             local_sem):
        my = lax.axis_index("i")
        right = lax.rem(my + 1, N_DEV)

        logits = jnp.dot(x_ref[:, :], w_ref[:, :],
                         preferred_element_type=jnp.float32)
        m_loc = jnp.max(logits, axis=1)
        e = jnp.exp(logits - m_loc[:, None])
        s_loc = jnp.sum(e, axis=1)
        chunk_ref[:, :] = e
        mystats_ref[0, :] = m_loc
        mystats_ref[1, :] = s_loc

        for dlt in range(1, N_DEV):
            tgt = lax.rem(my + dlt, N_DEV)
            pltpu.make_async_remote_copy(
                src_ref=mystats_ref,
                dst_ref=stats_ref.at[my],
                send_sem=stat_send_sems.at[dlt - 1],
                recv_sem=stat_recv_sems.at[dlt - 1],
                device_id=(tgt,),
                device_id_type=pl.DeviceIdType.MESH,
            ).start()
        cp = pltpu.make_async_copy(mystats_ref, stats_ref.at[my], local_sem)
        cp.start()
        cp.wait()
        for dlt in range(1, N_DEV):
            src = lax.rem(my - dlt + N_DEV, N_DEV)
            pltpu.make_async_remote_copy(
                src_ref=mystats_ref,
                dst_ref=stats_ref.at[src],
                send_sem=stat_send_sems.at[dlt - 1],
                recv_sem=stat_recv_sems.at[dlt - 1],
                device_id=(my,),
                device_id_type=pl.DeviceIdType.MESH,
            ).wait_recv()
        for dlt in range(1, N_DEV):
            tgt = lax.rem(my + dlt, N_DEV)
            pltpu.make_async_remote_copy(
                src_ref=mystats_ref,
                dst_ref=stats_ref.at[my],
                send_sem=stat_send_sems.at[dlt - 1],
                recv_sem=stat_recv_sems.at[dlt - 1],
                device_id=(tgt,),
                device_id_type=pl.DeviceIdType.MESH,
            ).wait_send()

        allm = stats_ref[:, 0, :]
        alls = stats_ref[:, 1, :]
        gm = jnp.max(allm, axis=0)
        gs = jnp.sum(alls * jnp.exp(allm - gm[None, :]), axis=0)
        scale = jnp.exp(m_loc - gm) / gs
        chunk_ref[:, :] = chunk_ref[:, :] * scale[:, None]

        cp2 = pltpu.make_async_copy(
            chunk_ref, out_ref.at[:, pl.ds(my * v_loc, v_loc)], local_sem)
        cp2.start()
        cp2.wait()

        for h in range(N_DEV - 1):
            o_s = lax.rem(my - h + 2 * N_DEV, N_DEV)
            o_r = lax.rem(my - h - 1 + 2 * N_DEV, N_DEV)
            send = pltpu.make_async_remote_copy(
                src_ref=out_ref.at[:, pl.ds(o_s * v_loc, v_loc)],
                dst_ref=out_ref.at[:, pl.ds(o_s * v_loc, v_loc)],
                send_sem=ring_send_sems.at[h],
                recv_sem=ring_recv_sems.at[h],
                device_id=(right,),
                device_id_type=pl.DeviceIdType.MESH,
            )
            send.start()
            send.wait_send()
            recv = pltpu.make_async_remote_copy(
                src_ref=out_ref.at[:, pl.ds(o_r * v_loc, v_loc)],
                dst_ref=out_ref.at[:, pl.ds(o_r * v_loc, v_loc)],
                send_sem=ring_send_sems.at[h],
                recv_sem=ring_recv_sems.at[h],
                device_id=(right,),
                device_id_type=pl.DeviceIdType.MESH,
            )
            recv.wait_recv()

    return pl.pallas_call(
        body,
        out_shape=jax.ShapeDtypeStruct((t, v_glob), jnp.float32),
        in_specs=[
            pl.BlockSpec(memory_space=pltpu.VMEM),
            pl.BlockSpec(memory_space=pltpu.VMEM),
        ],
        out_specs=pl.BlockSpec(memory_space=pl.ANY),
        scratch_shapes=[
            pltpu.VMEM((t, v_loc), jnp.float32),
            pltpu.VMEM((N_DEV, 2, t), jnp.float32),
            pltpu.VMEM((2, t), jnp.float32),
            pltpu.SemaphoreType.DMA((N_DEV - 1,)),
            pltpu.SemaphoreType.DMA((N_DEV - 1,)),
            pltpu.SemaphoreType.DMA((N_DEV - 1,)),
            pltpu.SemaphoreType.DMA((N_DEV - 1,)),
            pltpu.SemaphoreType.DMA,
        ],
    )(x, W)
